# baseline (speedup 1.0000x reference)
import functools

import jax
import jax.numpy as jnp
from jax import lax
from jax.experimental import pallas as pl
from jax.experimental.pallas import tpu as pltpu
from jax.experimental.pallas import tpu_sc as plsc

VOCAB = 1000000
D = 64
DP = 128
SCALE_F = 8.0
NC, NS = 2, 16
NW = NC * NS
L = 16

N_FULL = VOCAB // DP
TAIL0 = N_FULL * DP
TAILN = VOCAB - TAIL0
K_MAIN = 244
K_EXTRA_W = N_FULL - K_MAIN * NW

IDXW = 128
C = 128
N_SUB = C // IDXW
NBUF = 3


@functools.partial(
    pl.kernel,
    out_type=jax.ShapeDtypeStruct((VOCAB, DP), jnp.float32),
    mesh=plsc.VectorSubcoreMesh(core_axis_name="c", subcore_axis_name="s"),
    compiler_params=pltpu.CompilerParams(
        use_tc_tiling_on_sc=True, needs_layout_passes=False),
    scratch_types=[
        pltpu.VMEM((2, D, DP), jnp.float32),
        pltpu.VMEM((2, DP, DP), jnp.float32),
        pltpu.VMEM((TAILN, D), jnp.float32),
        [pltpu.SemaphoreType.DMA] * 2,
        [pltpu.SemaphoreType.DMA] * 2,
    ],
)
def _transpose_scale(tab_t, tail, t1m, bufin, bufout, buftail, isems, osems):
    wid = lax.axis_index("s") * NC + lax.axis_index("c")
    iota16 = lax.iota(jnp.int32, L)

    def src_col(k):
        return pl.multiple_of((wid + k * NW) * DP, DP)

    def fire_in(k, b):
        pltpu.async_copy(tab_t.at[:, pl.ds(src_col(k), DP)],
                         bufin.at[b], isems[b])

    def wait_in(b):
        pltpu.make_async_copy(tab_t.at[:, pl.ds(0, DP)], bufin.at[b],
                              isems[b]).wait()

    def transpose(b, nrows):
        for k in range(nrows // L):
            idx_r = iota16 + (L * k)

            @plsc.parallel_loop(0, D, unroll=8)
            def _(c):
                v = bufin[b, c, pl.ds(L * k, L)]
                plsc.store_scatter(
                    bufout.at[b], [idx_r, jnp.full((L,), c, jnp.int32)],
                    v * SCALE_F)

    def ocopy(k, b):
        pltpu.async_copy(bufout.at[b],
                         t1m.at[pl.ds(src_col(k), DP)], osems[b])

    def owait(b):
        pltpu.make_async_copy(bufout.at[b], t1m.at[pl.ds(0, DP)],
                              osems[b]).wait()

    fire_in(0, 0)
    wait_in(0)
    fire_in(1, 1)
    transpose(0, DP)
    ocopy(0, 0)
    wait_in(1)
    fire_in(2, 0)
    transpose(1, DP)
    ocopy(1, 1)

    def pair(t, carry):
        ka = 2 * t
        wait_in(0)
        fire_in(ka + 1, 1)
        owait(0)
        transpose(0, DP)
        ocopy(ka, 0)
        wait_in(1)
        fire_in(jnp.minimum(ka + 2, K_MAIN - 1), 0)
        owait(1)
        transpose(1, DP)
        ocopy(ka + 1, 1)
        return carry

    lax.fori_loop(1, K_MAIN // 2, pair, 0)
    wait_in(0)
    owait(0)
    owait(1)

    @pl.when(wid < K_EXTRA_W)
    def _():
        pltpu.sync_copy(tab_t.at[:, pl.ds(src_col(K_MAIN), DP)], bufin.at[0])
        transpose(0, DP)
        pltpu.sync_copy(bufout.at[0], t1m.at[pl.ds(src_col(K_MAIN), DP)])

    @pl.when(wid == NW - 1)
    def _():
        pltpu.sync_copy(tail, buftail)

        @plsc.parallel_loop(0, TAILN, unroll=4)
        def _(i):
            for jj in range(D // L):
                bufin[0, i, pl.ds(jj * L, L)] = (
                    buftail[i, pl.ds(jj * L, L)] * SCALE_F)

        pltpu.sync_copy(bufin.at[0, pl.ds(0, TAILN)],
                        t1m.at[pl.ds(TAIL0, TAILN)])


def _make_gather(B):
    b_per_w = B // NW
    n_chunks = b_per_w // C
    idx_rows_per_w = b_per_w // IDXW
    assert n_chunks >= 10 and (n_chunks - 2) % 6 == 0

    @functools.partial(
        pl.kernel,
        out_type=jax.ShapeDtypeStruct((B, D), jnp.float32),
        mesh=plsc.VectorSubcoreMesh(core_axis_name="c", subcore_axis_name="s"),
        compiler_params=pltpu.CompilerParams(
            use_tc_tiling_on_sc=True, needs_layout_passes=False),
        scratch_types=[
            pltpu.VMEM((idx_rows_per_w, IDXW), jnp.int32),
            pltpu.VMEM((NBUF, C, DP), jnp.float32),
            pltpu.VMEM((2, C, D), jnp.float32),
            [pltpu.SemaphoreType.DMA] * NBUF,
            [pltpu.SemaphoreType.DMA] * 2,
        ],
    )
    def gather(t1m, idx_hbm, out_hbm, idx_v, rows_v, packed_v, gsems, osems):
        wid = lax.axis_index("s") * NC + lax.axis_index("c")
        out_row0 = wid * b_per_w

        pltpu.sync_copy(idx_hbm.at[pl.ds(wid * idx_rows_per_w, idx_rows_per_w)],
                        idx_v)

        def fire(g, b):
            for j in range(N_SUB):
                pltpu.async_copy(t1m.at[idx_v.at[g * N_SUB + j]],
                                 rows_v.at[b, pl.ds(j * IDXW, IDXW)], gsems[b])

        def gwait(b):
            pltpu.make_async_copy(t1m.at[pl.ds(0, C)], rows_v.at[b],
                                  gsems[b]).wait()

        def pack(b, p):
            @plsc.parallel_loop(0, C, unroll=4)
            def _(i):
                for jj in range(D // L):
                    packed_v[p, i, pl.ds(jj * L, L)] = (
                        rows_v[b, i, pl.ds(jj * L, L)])

        def ocopy(g, p):
            base = pl.multiple_of(out_row0 + g * C, C)
            pltpu.async_copy(packed_v.at[p],
                             out_hbm.at[pl.ds(base, C)], osems[p])

        def owait(p):
            pltpu.make_async_copy(packed_v.at[0], out_hbm.at[pl.ds(0, C)],
                                  osems[p]).wait()

        def step(g, b, p, do_owait=True, do_fire=True):
            gwait(b)
            if do_owait:
                owait(p)
            pack(b, p)
            if do_fire:
                fire(g + 2, (b + 2) % NBUF)
            ocopy(g, p)

        fire(0, 0)
        fire(1, 1)
        step(0, 0, 0, do_owait=False)
        step(1, 1, 1, do_owait=False)

        def sextet(t, carry):
            g0 = 6 * t + 2
            for u in range(6):
                g = g0 + u
                step(g, (2 + u) % NBUF, u % 2)
            return carry

        lax.fori_loop(0, (n_chunks - 8) // 6, sextet, 0)

        n = n_chunks
        for g in (n - 6, n - 5, n - 4, n - 3):
            step(g, g % NBUF, g % 2)
        step(n - 2, (n - 2) % NBUF, (n - 2) % 2, do_fire=False)
        step(n - 1, (n - 1) % NBUF, (n - 1) % 2, do_fire=False)
        owait((n - 2) % 2)
        owait((n - 1) % 2)

    return gather


def kernel(x, table):
    B = x.shape[0] * x.shape[1]
    idx = x.reshape(B // IDXW, IDXW).astype(jnp.int32)
    t1m = _transpose_scale(table.T, table[TAIL0:])
    out = _make_gather(B)(t1m, idx)
    return out.reshape(x.shape + (D,))


# speedup vs baseline: 1.0976x; 1.0976x over previous
import functools

import jax
import jax.numpy as jnp
from jax import lax
from jax.experimental import pallas as pl
from jax.experimental.pallas import tpu as pltpu
from jax.experimental.pallas import tpu_sc as plsc

VOCAB = 1000000
D = 64
DP = 128
SCALE_F = 8.0
NC, NS = 2, 16
NW = NC * NS
L = 16

N_FULL = VOCAB // DP
TAIL0 = N_FULL * DP
TAILN = VOCAB - TAIL0
K_MAIN = 244
K_EXTRA_W = N_FULL - K_MAIN * NW

IDXW = 128
C = 128
N_SUB = C // IDXW
NBUF = 3


@functools.partial(
    pl.kernel,
    out_type=jax.ShapeDtypeStruct((VOCAB, DP), jnp.float32),
    mesh=plsc.VectorSubcoreMesh(core_axis_name="c", subcore_axis_name="s"),
    compiler_params=pltpu.CompilerParams(
        use_tc_tiling_on_sc=True, needs_layout_passes=False),
    scratch_types=[
        pltpu.VMEM((2, D, DP), jnp.float32),
        pltpu.VMEM((2, DP, DP), jnp.float32),
        pltpu.VMEM((TAILN, D), jnp.float32),
        [pltpu.SemaphoreType.DMA] * 2,
        [pltpu.SemaphoreType.DMA] * 2,
    ],
)
def _transpose_scale(tab_t, tail, t1m, bufin, bufout, buftail, isems, osems):
    wid = lax.axis_index("s") * NC + lax.axis_index("c")
    iota16 = lax.iota(jnp.int32, L)

    def src_col(k):
        return pl.multiple_of((wid + k * NW) * DP, DP)

    def fire_in(k, b):
        pltpu.async_copy(tab_t.at[:, pl.ds(src_col(k), DP)],
                         bufin.at[b], isems[b])

    def wait_in(b):
        pltpu.make_async_copy(tab_t.at[:, pl.ds(0, DP)], bufin.at[b],
                              isems[b]).wait()

    qvecs = [iota16 + L * q for q in range(D // L)]

    def transpose(b, nrows):
        @plsc.parallel_loop(0, nrows, unroll=8)
        def _(r):
            r_vec = jnp.full((L,), r, jnp.int32)
            for q in range(D // L):
                v = plsc.load_gather(bufin.at[b], [qvecs[q], r_vec])
                bufout[b, r, pl.ds(L * q, L)] = v * SCALE_F

    def ocopy(k, b):
        pltpu.async_copy(bufout.at[b],
                         t1m.at[pl.ds(src_col(k), DP)], osems[b])

    def owait(b):
        pltpu.make_async_copy(bufout.at[b], t1m.at[pl.ds(0, DP)],
                              osems[b]).wait()

    fire_in(0, 0)
    wait_in(0)
    fire_in(1, 1)
    transpose(0, DP)
    ocopy(0, 0)
    wait_in(1)
    fire_in(2, 0)
    transpose(1, DP)
    ocopy(1, 1)

    def pair(t, carry):
        ka = 2 * t
        wait_in(0)
        fire_in(ka + 1, 1)
        owait(0)
        transpose(0, DP)
        ocopy(ka, 0)
        wait_in(1)
        fire_in(jnp.minimum(ka + 2, K_MAIN - 1), 0)
        owait(1)
        transpose(1, DP)
        ocopy(ka + 1, 1)
        return carry

    lax.fori_loop(1, K_MAIN // 2, pair, 0)
    wait_in(0)
    owait(0)
    owait(1)

    @pl.when(wid < K_EXTRA_W)
    def _():
        pltpu.sync_copy(tab_t.at[:, pl.ds(src_col(K_MAIN), DP)], bufin.at[0])
        transpose(0, DP)
        pltpu.sync_copy(bufout.at[0], t1m.at[pl.ds(src_col(K_MAIN), DP)])

    @pl.when(wid == NW - 1)
    def _():
        pltpu.sync_copy(tail, buftail)

        @plsc.parallel_loop(0, TAILN, unroll=4)
        def _(i):
            for jj in range(D // L):
                bufin[0, i, pl.ds(jj * L, L)] = (
                    buftail[i, pl.ds(jj * L, L)] * SCALE_F)

        pltpu.sync_copy(bufin.at[0, pl.ds(0, TAILN)],
                        t1m.at[pl.ds(TAIL0, TAILN)])


def _make_gather(B):
    b_per_w = B // NW
    n_chunks = b_per_w // C
    idx_rows_per_w = b_per_w // IDXW
    assert n_chunks >= 10 and (n_chunks - 2) % 6 == 0

    @functools.partial(
        pl.kernel,
        out_type=jax.ShapeDtypeStruct((B, D), jnp.float32),
        mesh=plsc.VectorSubcoreMesh(core_axis_name="c", subcore_axis_name="s"),
        compiler_params=pltpu.CompilerParams(
            use_tc_tiling_on_sc=True, needs_layout_passes=False),
        scratch_types=[
            pltpu.VMEM((idx_rows_per_w, IDXW), jnp.int32),
            pltpu.VMEM((NBUF, C, DP), jnp.float32),
            pltpu.VMEM((2, C, D), jnp.float32),
            [pltpu.SemaphoreType.DMA] * NBUF,
            [pltpu.SemaphoreType.DMA] * 2,
        ],
    )
    def gather(t1m, idx_hbm, out_hbm, idx_v, rows_v, packed_v, gsems, osems):
        wid = lax.axis_index("s") * NC + lax.axis_index("c")
        out_row0 = wid * b_per_w

        pltpu.sync_copy(idx_hbm.at[pl.ds(wid * idx_rows_per_w, idx_rows_per_w)],
                        idx_v)

        def fire(g, b):
            for j in range(N_SUB):
                pltpu.async_copy(t1m.at[idx_v.at[g * N_SUB + j]],
                                 rows_v.at[b, pl.ds(j * IDXW, IDXW)], gsems[b])

        def gwait(b):
            pltpu.make_async_copy(t1m.at[pl.ds(0, C)], rows_v.at[b],
                                  gsems[b]).wait()

        def pack(b, p):
            @plsc.parallel_loop(0, C, unroll=4)
            def _(i):
                for jj in range(D // L):
                    packed_v[p, i, pl.ds(jj * L, L)] = (
                        rows_v[b, i, pl.ds(jj * L, L)])

        def ocopy(g, p):
            base = pl.multiple_of(out_row0 + g * C, C)
            pltpu.async_copy(packed_v.at[p],
                             out_hbm.at[pl.ds(base, C)], osems[p])

        def owait(p):
            pltpu.make_async_copy(packed_v.at[0], out_hbm.at[pl.ds(0, C)],
                                  osems[p]).wait()

        def step(g, b, p, do_owait=True, do_fire=True):
            gwait(b)
            if do_owait:
                owait(p)
            pack(b, p)
            if do_fire:
                fire(g + 2, (b + 2) % NBUF)
            ocopy(g, p)

        fire(0, 0)
        fire(1, 1)
        step(0, 0, 0, do_owait=False)
        step(1, 1, 1, do_owait=False)

        def sextet(t, carry):
            g0 = 6 * t + 2
            for u in range(6):
                g = g0 + u
                step(g, (2 + u) % NBUF, u % 2)
            return carry

        lax.fori_loop(0, (n_chunks - 8) // 6, sextet, 0)

        n = n_chunks
        for g in (n - 6, n - 5, n - 4, n - 3):
            step(g, g % NBUF, g % 2)
        step(n - 2, (n - 2) % NBUF, (n - 2) % 2, do_fire=False)
        step(n - 1, (n - 1) % NBUF, (n - 1) % 2, do_fire=False)
        owait((n - 2) % 2)
        owait((n - 1) % 2)

    return gather


def kernel(x, table):
    B = x.shape[0] * x.shape[1]
    idx = x.reshape(B // IDXW, IDXW).astype(jnp.int32)
    t1m = _transpose_scale(table.T, table[TAIL0:])
    out = _make_gather(B)(t1m, idx)
    return out.reshape(x.shape + (D,))


# speedup vs baseline: 1.7036x; 1.5521x over previous
import functools

import jax
import jax.numpy as jnp
from jax import lax
from jax.experimental import pallas as pl
from jax.experimental.pallas import tpu as pltpu
from jax.experimental.pallas import tpu_sc as plsc

VOCAB = 1000000
D = 64
DP = 128
SCALE_F = 8.0
NC, NS = 2, 16
NW = NC * NS
L = 16

N_FULL = VOCAB // DP
TAIL0 = N_FULL * DP
TAILN = VOCAB - TAIL0
K_MAIN = 244
K_EXTRA_W = N_FULL - K_MAIN * NW

IDXW = 128
C = 128
N_SUB = C // IDXW
NBUF = 3


@functools.partial(
    pl.kernel,
    out_type=jax.ShapeDtypeStruct((VOCAB, DP), jnp.float32),
    mesh=plsc.VectorSubcoreMesh(core_axis_name="c", subcore_axis_name="s"),
    compiler_params=pltpu.CompilerParams(
        use_tc_tiling_on_sc=True, needs_layout_passes=False),
    scratch_types=[
        pltpu.VMEM((2, D, DP), jnp.float32),
        pltpu.VMEM((2, DP, DP), jnp.float32),
        pltpu.VMEM((TAILN, D), jnp.float32),
        [pltpu.SemaphoreType.DMA] * 2,
        [pltpu.SemaphoreType.DMA] * 2,
    ],
)
def _transpose_scale(tab_t, tail, t1m, bufin, bufout, buftail, isems, osems):
    wid = lax.axis_index("s") * NC + lax.axis_index("c")
    iota16 = lax.iota(jnp.int32, L)

    def src_col(k):
        return pl.multiple_of((wid + k * NW) * DP, DP)

    def fire_in(k, b):
        pltpu.async_copy(tab_t.at[:, pl.ds(src_col(k), DP)],
                         bufin.at[b], isems[b])

    def wait_in(b):
        pltpu.make_async_copy(tab_t.at[:, pl.ds(0, DP)], bufin.at[b],
                              isems[b]).wait()

    zeros16 = jnp.zeros((L,), jnp.int32)
    fpat = [((iota16 + s) % L) * DP + iota16 for s in range(L)]
    gpat = [iota16 * DP + ((iota16 + s) % L) for s in range(L)]

    def transpose(b, nrows):
        nsub = (nrows // L) * (D // L)

        @plsc.parallel_loop(0, nsub, unroll=2)
        def _(t2):
            r16 = (t2 // (D // L)) * L
            q16 = (t2 % (D // L)) * L
            off_l = q16 * DP + r16
            off_s = r16 * DP + q16
            for s in range(L):
                v = plsc.load_gather(bufin.at[b], [zeros16, fpat[s] + off_l])
                plsc.store_scatter(bufout.at[b], [zeros16, gpat[s] + off_s],
                                   v * SCALE_F)

    def ocopy(k, b):
        pltpu.async_copy(bufout.at[b],
                         t1m.at[pl.ds(src_col(k), DP)], osems[b])

    def owait(b):
        pltpu.make_async_copy(bufout.at[b], t1m.at[pl.ds(0, DP)],
                              osems[b]).wait()

    fire_in(0, 0)
    wait_in(0)
    fire_in(1, 1)
    transpose(0, DP)
    ocopy(0, 0)
    wait_in(1)
    fire_in(2, 0)
    transpose(1, DP)
    ocopy(1, 1)

    def pair(t, carry):
        ka = 2 * t
        wait_in(0)
        fire_in(ka + 1, 1)
        owait(0)
        transpose(0, DP)
        ocopy(ka, 0)
        wait_in(1)
        fire_in(jnp.minimum(ka + 2, K_MAIN - 1), 0)
        owait(1)
        transpose(1, DP)
        ocopy(ka + 1, 1)
        return carry

    lax.fori_loop(1, K_MAIN // 2, pair, 0)
    wait_in(0)
    owait(0)
    owait(1)

    @pl.when(wid < K_EXTRA_W)
    def _():
        pltpu.sync_copy(tab_t.at[:, pl.ds(src_col(K_MAIN), DP)], bufin.at[0])
        transpose(0, DP)
        pltpu.sync_copy(bufout.at[0], t1m.at[pl.ds(src_col(K_MAIN), DP)])

    @pl.when(wid == NW - 1)
    def _():
        pltpu.sync_copy(tail, buftail)

        @plsc.parallel_loop(0, TAILN, unroll=4)
        def _(i):
            for jj in range(D // L):
                bufin[0, i, pl.ds(jj * L, L)] = (
                    buftail[i, pl.ds(jj * L, L)] * SCALE_F)

        pltpu.sync_copy(bufin.at[0, pl.ds(0, TAILN)],
                        t1m.at[pl.ds(TAIL0, TAILN)])


def _make_gather(B):
    b_per_w = B // NW
    n_chunks = b_per_w // C
    idx_rows_per_w = b_per_w // IDXW
    assert n_chunks >= 10 and (n_chunks - 2) % 6 == 0

    @functools.partial(
        pl.kernel,
        out_type=jax.ShapeDtypeStruct((B, D), jnp.float32),
        mesh=plsc.VectorSubcoreMesh(core_axis_name="c", subcore_axis_name="s"),
        compiler_params=pltpu.CompilerParams(
            use_tc_tiling_on_sc=True, needs_layout_passes=False),
        scratch_types=[
            pltpu.VMEM((idx_rows_per_w, IDXW), jnp.int32),
            pltpu.VMEM((NBUF, C, DP), jnp.float32),
            pltpu.VMEM((2, C, D), jnp.float32),
            [pltpu.SemaphoreType.DMA] * NBUF,
            [pltpu.SemaphoreType.DMA] * 2,
        ],
    )
    def gather(t1m, idx_hbm, out_hbm, idx_v, rows_v, packed_v, gsems, osems):
        wid = lax.axis_index("s") * NC + lax.axis_index("c")
        out_row0 = wid * b_per_w

        pltpu.sync_copy(idx_hbm.at[pl.ds(wid * idx_rows_per_w, idx_rows_per_w)],
                        idx_v)

        def fire(g, b):
            for j in range(N_SUB):
                pltpu.async_copy(t1m.at[idx_v.at[g * N_SUB + j]],
                                 rows_v.at[b, pl.ds(j * IDXW, IDXW)], gsems[b])

        def gwait(b):
            pltpu.make_async_copy(t1m.at[pl.ds(0, C)], rows_v.at[b],
                                  gsems[b]).wait()

        def pack(b, p):
            @plsc.parallel_loop(0, C, unroll=4)
            def _(i):
                for jj in range(D // L):
                    packed_v[p, i, pl.ds(jj * L, L)] = (
                        rows_v[b, i, pl.ds(jj * L, L)])

        def ocopy(g, p):
            base = pl.multiple_of(out_row0 + g * C, C)
            pltpu.async_copy(packed_v.at[p],
                             out_hbm.at[pl.ds(base, C)], osems[p])

        def owait(p):
            pltpu.make_async_copy(packed_v.at[0], out_hbm.at[pl.ds(0, C)],
                                  osems[p]).wait()

        def step(g, b, p, do_owait=True, do_fire=True):
            gwait(b)
            if do_owait:
                owait(p)
            pack(b, p)
            if do_fire:
                fire(g + 2, (b + 2) % NBUF)
            ocopy(g, p)

        fire(0, 0)
        fire(1, 1)
        step(0, 0, 0, do_owait=False)
        step(1, 1, 1, do_owait=False)

        def sextet(t, carry):
            g0 = 6 * t + 2
            for u in range(6):
                g = g0 + u
                step(g, (2 + u) % NBUF, u % 2)
            return carry

        lax.fori_loop(0, (n_chunks - 8) // 6, sextet, 0)

        n = n_chunks
        for g in (n - 6, n - 5, n - 4, n - 3):
            step(g, g % NBUF, g % 2)
        step(n - 2, (n - 2) % NBUF, (n - 2) % 2, do_fire=False)
        step(n - 1, (n - 1) % NBUF, (n - 1) % 2, do_fire=False)
        owait((n - 2) % 2)
        owait((n - 1) % 2)

    return gather


def kernel(x, table):
    B = x.shape[0] * x.shape[1]
    idx = x.reshape(B // IDXW, IDXW).astype(jnp.int32)
    t1m = _transpose_scale(table.T, table[TAIL0:])
    out = _make_gather(B)(t1m, idx)
    return out.reshape(x.shape + (D,))
